# Initial kernel scaffold; baseline (speedup 1.0000x reference)
import jax
import jax.numpy as jnp
from jax.experimental import pallas as pl


def kernel(feature_table, nodes, neigh_index, feature_dim):
    raise NotImplementedError("write your pallas kernel here")


# speedup vs baseline: 3.2403x; 3.2403x over previous
import functools

import jax
import jax.numpy as jnp
from jax import lax
from jax.experimental import pallas as pl
from jax.experimental.pallas import tpu as pltpu
from jax.experimental.pallas import tpu_sc as plsc

N_NODES = 100000
BATCH = 10000
DEG = 32
D = 128
G = 4
IDX_PER_G = G * DEG
NGROUPS = BATCH // G


@functools.lru_cache(maxsize=1)
def _build():
    info = plsc.get_sparse_core_info()
    NC, NS, L = info.num_cores, info.num_subcores, info.num_lanes
    NW = NC * NS
    NV = D // L
    K = -(-NGROUPS // NW)

    mesh = plsc.VectorSubcoreMesh(core_axis_name="c", subcore_axis_name="s")

    @functools.partial(
        pl.kernel,
        mesh=mesh,
        out_type=jax.ShapeDtypeStruct((BATCH, D), jnp.float32),
        scratch_types=[
            pltpu.VMEM((IDX_PER_G,), jnp.int32),
            pltpu.VMEM((IDX_PER_G, D), jnp.float32),
            pltpu.VMEM((G, D), jnp.float32),
            pltpu.VMEM((D,), jnp.float32),
            pltpu.SemaphoreType.DMA,
        ],
        compiler_params=pltpu.CompilerParams(needs_layout_passes=False),
    )
    def agg(table_hbm, neigh_hbm, out_hbm, idx_v, rows_v, out_v, row0_v, sem):
        wid = lax.axis_index("s") * NC + lax.axis_index("c")
        pltpu.sync_copy(table_hbm.at[0], row0_v)

        def step(k, carry):
            g = wid + NW * k

            @pl.when(g < NGROUPS)
            def _():
                pltpu.sync_copy(
                    neigh_hbm.at[pl.ds(g * IDX_PER_G, IDX_PER_G)], idx_v)
                pltpu.async_copy(table_hbm.at[idx_v], rows_v, sem).wait()
                for r in range(G):
                    i0 = idx_v[pl.ds(r * DEG, L)]
                    i1 = idx_v[pl.ds(r * DEG + L, L)]
                    nz_s = jnp.sum(jnp.where(i0 == 0, 1.0, 0.0)
                                   + jnp.where(i1 == 0, 1.0, 0.0))
                    nzf = jnp.full((L,), nz_s, dtype=jnp.float32)

                    def body(n, acc):
                        row = r * DEG + n
                        return tuple(acc[v] + rows_v[row, pl.ds(v * L, L)]
                                     for v in range(NV))

                    acc = lax.fori_loop(
                        0, DEG, body,
                        tuple(jnp.zeros((L,), jnp.float32) for _ in range(NV)))
                    cnt = jnp.float32(DEG) - nzf
                    cnt = jnp.where(cnt == 0.0, 1.0, cnt)
                    scale = 1.0 / cnt
                    for v in range(NV):
                        out_v[r, pl.ds(v * L, L)] = (
                            acc[v] - nzf * row0_v[pl.ds(v * L, L)]) * scale
                pltpu.sync_copy(out_v, out_hbm.at[pl.ds(g * G, G)])

            return carry

        lax.fori_loop(0, K, step, 0)

    return agg


def kernel(feature_table, nodes, neigh_index, feature_dim):
    del nodes, feature_dim
    neigh_flat = neigh_index.reshape(-1).astype(jnp.int32)
    return _build()(feature_table, neigh_flat)


# speedup vs baseline: 5.6622x; 1.7474x over previous
import functools

import jax
import jax.numpy as jnp
from jax import lax
from jax.experimental import pallas as pl
from jax.experimental.pallas import tpu as pltpu
from jax.experimental.pallas import tpu_sc as plsc

N_NODES = 100000
BATCH = 10000
DEG = 32
D = 128
G = 4
IDX_PER_G = G * DEG
NGROUPS = BATCH // G
NBUF = 2


@functools.lru_cache(maxsize=1)
def _build():
    info = plsc.get_sparse_core_info()
    NC, NS, L = info.num_cores, info.num_subcores, info.num_lanes
    NW = NC * NS
    NV = D // L
    K = -(-NGROUPS // NW)
    K_PAD = -(-K // NBUF) * NBUF

    mesh = plsc.VectorSubcoreMesh(core_axis_name="c", subcore_axis_name="s")

    scratch = []
    for _ in range(NBUF):
        scratch += [
            pltpu.VMEM((IDX_PER_G,), jnp.int32),
            pltpu.VMEM((IDX_PER_G, D), jnp.float32),
            pltpu.VMEM((G, D), jnp.float32),
            pltpu.SemaphoreType.DMA,
            pltpu.SemaphoreType.DMA,
        ]
    scratch.append(pltpu.VMEM((D,), jnp.float32))

    @functools.partial(
        pl.kernel,
        mesh=mesh,
        out_type=jax.ShapeDtypeStruct((BATCH, D), jnp.float32),
        scratch_types=scratch,
        compiler_params=pltpu.CompilerParams(needs_layout_passes=False),
    )
    def agg(table_hbm, neigh_hbm, out_hbm, *refs):
        bufs = [refs[5 * b:5 * b + 5] for b in range(NBUF)]
        row0_v = refs[5 * NBUF]
        wid = lax.axis_index("s") * NC + lax.axis_index("c")
        pltpu.sync_copy(table_hbm.at[0], row0_v)

        def start_fetch(g, idx_v, rows_v, sem):
            @pl.when(g < NGROUPS)
            def _():
                pltpu.sync_copy(
                    neigh_hbm.at[pl.ds(g * IDX_PER_G, IDX_PER_G)], idx_v)
                pltpu.async_copy(table_hbm.at[idx_v], rows_v, sem)

        for b in range(NBUF):
            start_fetch(wid + NW * b, *bufs[b][:2], bufs[b][3])

        def step(i, carry):
            for b in range(NBUF):
                idx_v, rows_v, out_v, sem, sem_o = bufs[b]
                k = i * NBUF + b
                g = wid + NW * k

                @pl.when(g < NGROUPS)
                def _(idx_v=idx_v, rows_v=rows_v, out_v=out_v, sem=sem,
                      sem_o=sem_o, g=g):
                    pltpu.make_async_copy(
                        table_hbm.at[idx_v], rows_v, sem).wait()

                    @pl.when(g >= NW * NBUF)
                    def _():
                        pltpu.make_async_copy(
                            out_v, out_hbm.at[pl.ds(0, G)], sem_o).wait()

                    for r in range(G):
                        i0 = idx_v[pl.ds(r * DEG, L)]
                        i1 = idx_v[pl.ds(r * DEG + L, L)]
                        nz_s = jnp.sum(jnp.where(i0 == 0, 1.0, 0.0)
                                       + jnp.where(i1 == 0, 1.0, 0.0))
                        nzf = jnp.full((L,), nz_s, dtype=jnp.float32)

                        def body(n2, acc):
                            row = r * DEG + 2 * n2
                            acc = tuple(
                                acc[v] + rows_v[row, pl.ds(v * L, L)]
                                for v in range(NV))
                            return tuple(
                                acc[v] + rows_v[row + 1, pl.ds(v * L, L)]
                                for v in range(NV))

                        acc = lax.fori_loop(
                            0, DEG // 2, body,
                            tuple(jnp.zeros((L,), jnp.float32)
                                  for _ in range(NV)))
                        cnt = jnp.float32(DEG) - nzf
                        cnt = jnp.where(cnt == 0.0, 1.0, cnt)
                        scale = 1.0 / cnt
                        for v in range(NV):
                            out_v[r, pl.ds(v * L, L)] = (
                                acc[v] - nzf * row0_v[pl.ds(v * L, L)]) * scale

                    pltpu.async_copy(
                        out_v, out_hbm.at[pl.ds(g * G, G)], sem_o)
                    start_fetch(g + NW * NBUF, idx_v, rows_v, sem)

            return carry

        lax.fori_loop(0, K_PAD // NBUF, step, 0)

        for b in range(NBUF):
            pltpu.make_async_copy(
                bufs[b][2], out_hbm.at[pl.ds(0, G)], bufs[b][4]).wait()

    return agg


def kernel(feature_table, nodes, neigh_index, feature_dim):
    del nodes, feature_dim
    neigh_flat = neigh_index.reshape(-1).astype(jnp.int32)
    return _build()(feature_table, neigh_flat)


# speedup vs baseline: 6.4163x; 1.1332x over previous
import functools

import jax
import jax.numpy as jnp
from jax import lax
from jax.experimental import pallas as pl
from jax.experimental.pallas import tpu as pltpu
from jax.experimental.pallas import tpu_sc as plsc

N_NODES = 100000
BATCH = 10000
DEG = 32
D = 128
G = 4
IDX_PER_G = G * DEG
NGROUPS = BATCH // G
NBUF = 2


@functools.lru_cache(maxsize=1)
def _build():
    info = plsc.get_sparse_core_info()
    NC, NS, L = info.num_cores, info.num_subcores, info.num_lanes
    NW = NC * NS
    NV = D // L
    KLO = NGROUPS // NW
    NHI = NGROUPS - KLO * NW
    K = KLO + 1
    K_PAD = -(-K // NBUF) * NBUF
    NG_PAD = NW * KLO + NHI + K

    mesh = plsc.VectorSubcoreMesh(core_axis_name="c", subcore_axis_name="s")

    scratch = [pltpu.VMEM((K * IDX_PER_G,), jnp.int32)]
    for _ in range(NBUF):
        scratch += [
            pltpu.VMEM((IDX_PER_G, D), jnp.float32),
            pltpu.VMEM((G, D), jnp.float32),
            pltpu.SemaphoreType.DMA,
            pltpu.SemaphoreType.DMA,
        ]
    scratch.append(pltpu.VMEM((D,), jnp.float32))

    @functools.partial(
        pl.kernel,
        mesh=mesh,
        out_type=jax.ShapeDtypeStruct((BATCH, D), jnp.float32),
        scratch_types=scratch,
        compiler_params=pltpu.CompilerParams(needs_layout_passes=False),
    )
    def agg(table_hbm, neigh_hbm, out_hbm, idx_all, *refs):
        bufs = [refs[4 * b:4 * b + 4] for b in range(NBUF)]
        row0_v = refs[4 * NBUF]
        wid = lax.axis_index("s") * NC + lax.axis_index("c")
        kw = jnp.where(wid < NHI, KLO + 1, KLO)
        g0 = wid * KLO + jnp.minimum(wid, NHI)

        pltpu.sync_copy(table_hbm.at[0], row0_v)
        pltpu.sync_copy(
            neigh_hbm.at[pl.ds(g0 * IDX_PER_G, K * IDX_PER_G)], idx_all)

        def start_fetch(j, rows_v, sem):
            @pl.when(j < kw)
            def _():
                pltpu.async_copy(
                    table_hbm.at[idx_all.at[pl.ds(j * IDX_PER_G, IDX_PER_G)]],
                    rows_v, sem)

        for b in range(NBUF):
            start_fetch(b, bufs[b][0], bufs[b][2])

        def step(i, carry):
            for b in range(NBUF):
                rows_v, out_v, sem, sem_o = bufs[b]
                j = i * NBUF + b

                @pl.when(j < kw)
                def _(rows_v=rows_v, out_v=out_v, sem=sem, sem_o=sem_o, j=j):
                    pltpu.make_async_copy(
                        table_hbm.at[idx_all.at[pl.ds(j * IDX_PER_G,
                                                      IDX_PER_G)]],
                        rows_v, sem).wait()

                    @pl.when(j >= NBUF)
                    def _():
                        pltpu.make_async_copy(
                            out_v, out_hbm.at[pl.ds(0, G)], sem_o).wait()

                    for r in range(G):
                        i0 = idx_all[pl.ds(j * IDX_PER_G + r * DEG, L)]
                        i1 = idx_all[pl.ds(j * IDX_PER_G + r * DEG + L, L)]
                        nz_s = jnp.sum(jnp.where(i0 == 0, 1.0, 0.0)
                                       + jnp.where(i1 == 0, 1.0, 0.0))
                        nzf = jnp.full((L,), nz_s, dtype=jnp.float32)

                        def body(n2, acc):
                            row = r * DEG + 2 * n2
                            acc = tuple(
                                acc[v] + rows_v[row, pl.ds(v * L, L)]
                                for v in range(NV))
                            return tuple(
                                acc[v] + rows_v[row + 1, pl.ds(v * L, L)]
                                for v in range(NV))

                        acc = lax.fori_loop(
                            0, DEG // 2, body,
                            tuple(jnp.zeros((L,), jnp.float32)
                                  for _ in range(NV)))
                        cnt = jnp.float32(DEG) - nzf
                        cnt = jnp.where(cnt == 0.0, 1.0, cnt)
                        scale = 1.0 / cnt
                        for v in range(NV):
                            out_v[r, pl.ds(v * L, L)] = (
                                acc[v] - nzf * row0_v[pl.ds(v * L, L)]) * scale

                    pltpu.async_copy(
                        out_v, out_hbm.at[pl.ds((g0 + j) * G, G)], sem_o)
                    start_fetch(j + NBUF, rows_v, sem)

            return carry

        lax.fori_loop(0, K_PAD // NBUF, step, 0)

        for b in range(NBUF):
            pltpu.make_async_copy(
                bufs[b][1], out_hbm.at[pl.ds(0, G)], bufs[b][3]).wait()

    def run(feature_table, neigh_flat):
        pad = jnp.zeros(((NG_PAD - NGROUPS) * IDX_PER_G,), jnp.int32)
        neigh_flat = jnp.concatenate([neigh_flat, pad], axis=0)
        return agg(feature_table, neigh_flat)

    return run


def kernel(feature_table, nodes, neigh_index, feature_dim):
    del nodes, feature_dim
    neigh_flat = neigh_index.reshape(-1).astype(jnp.int32)
    return _build()(feature_table, neigh_flat)


# speedup vs baseline: 8.1912x; 1.2766x over previous
import functools

import jax
import jax.numpy as jnp
from jax import lax
from jax.experimental import pallas as pl
from jax.experimental.pallas import tpu as pltpu
from jax.experimental.pallas import tpu_sc as plsc

N_NODES = 100000
BATCH = 10000
DEG = 32
D = 128
G = 4
IDX_PER_G = G * DEG
NGROUPS = BATCH // G
NBUF = 3


@functools.lru_cache(maxsize=1)
def _build():
    info = plsc.get_sparse_core_info()
    NC, NS, L = info.num_cores, info.num_subcores, info.num_lanes
    NW = NC * NS
    NV = D // L
    KLO = NGROUPS // NW
    NHI = NGROUPS - KLO * NW
    K = KLO + 1
    K_PAD = -(-K // NBUF) * NBUF

    mesh = plsc.VectorSubcoreMesh(core_axis_name="c", subcore_axis_name="s")

    scratch = [pltpu.VMEM((K * IDX_PER_G,), jnp.int32)]
    for _ in range(NBUF):
        scratch += [
            pltpu.VMEM((IDX_PER_G, D), jnp.float32),
            pltpu.VMEM((G, D), jnp.float32),
            pltpu.SemaphoreType.DMA,
            pltpu.SemaphoreType.DMA,
        ]
    scratch.append(pltpu.VMEM((D,), jnp.float32))

    @functools.partial(
        pl.kernel,
        mesh=mesh,
        out_type=jax.ShapeDtypeStruct((BATCH, D), jnp.float32),
        scratch_types=scratch,
        compiler_params=pltpu.CompilerParams(needs_layout_passes=False),
    )
    def agg(table_hbm, neigh_hbm, out_hbm, idx_all, *refs):
        bufs = [refs[4 * b:4 * b + 4] for b in range(NBUF)]
        row0_v = refs[4 * NBUF]
        wid = lax.axis_index("s") * NC + lax.axis_index("c")
        kw = jnp.where(wid < NHI, KLO + 1, KLO)
        g0 = wid * KLO + jnp.minimum(wid, NHI)
        gs = jnp.minimum(g0, NGROUPS - K)
        ofs = (g0 - gs) * IDX_PER_G

        pltpu.sync_copy(table_hbm.at[0], row0_v)
        pltpu.sync_copy(
            neigh_hbm.at[pl.ds(gs * IDX_PER_G, K * IDX_PER_G)], idx_all)

        def start_fetch(j, rows_v, sem):
            @pl.when(j < kw)
            def _():
                pltpu.async_copy(
                    table_hbm.at[
                        idx_all.at[pl.ds(ofs + j * IDX_PER_G, IDX_PER_G)]],
                    rows_v, sem)

        for b in range(NBUF):
            start_fetch(b, bufs[b][0], bufs[b][2])

        def step(i, carry):
            for b in range(NBUF):
                rows_v, out_v, sem, sem_o = bufs[b]
                j = i * NBUF + b

                @pl.when(j < kw)
                def _(rows_v=rows_v, out_v=out_v, sem=sem, sem_o=sem_o, j=j):
                    pltpu.make_async_copy(
                        table_hbm.at[idx_all.at[pl.ds(ofs + j * IDX_PER_G,
                                                      IDX_PER_G)]],
                        rows_v, sem).wait()

                    @pl.when(j >= NBUF)
                    def _():
                        pltpu.make_async_copy(
                            out_v, out_hbm.at[pl.ds(0, G)], sem_o).wait()

                    for r in range(G):
                        i0 = idx_all[pl.ds(ofs + j * IDX_PER_G + r * DEG, L)]
                        i1 = idx_all[pl.ds(ofs + j * IDX_PER_G + r * DEG + L,
                                           L)]
                        nz_s = jnp.sum(jnp.where(i0 == 0, 1.0, 0.0)
                                       + jnp.where(i1 == 0, 1.0, 0.0))
                        nzf = jnp.full((L,), nz_s, dtype=jnp.float32)

                        def body(n2, acc):
                            row = r * DEG + 2 * n2
                            acc = tuple(
                                acc[v] + rows_v[row, pl.ds(v * L, L)]
                                for v in range(NV))
                            return tuple(
                                acc[v] + rows_v[row + 1, pl.ds(v * L, L)]
                                for v in range(NV))

                        acc = lax.fori_loop(
                            0, DEG // 2, body,
                            tuple(jnp.zeros((L,), jnp.float32)
                                  for _ in range(NV)))
                        cnt = jnp.float32(DEG) - nzf
                        cnt = jnp.where(cnt == 0.0, 1.0, cnt)
                        scale = 1.0 / cnt
                        for v in range(NV):
                            out_v[r, pl.ds(v * L, L)] = (
                                acc[v] - nzf * row0_v[pl.ds(v * L, L)]) * scale

                    pltpu.async_copy(
                        out_v, out_hbm.at[pl.ds((g0 + j) * G, G)], sem_o)
                    start_fetch(j + NBUF, rows_v, sem)

            return carry

        lax.fori_loop(0, K_PAD // NBUF, step, 0)

        for b in range(NBUF):
            pltpu.make_async_copy(
                bufs[b][1], out_hbm.at[pl.ds(0, G)], bufs[b][3]).wait()

    return agg


def kernel(feature_table, nodes, neigh_index, feature_dim):
    del nodes, feature_dim
    neigh_flat = neigh_index.reshape(-1).astype(jnp.int32)
    return _build()(feature_table, neigh_flat)
